# Initial kernel scaffold; baseline (speedup 1.0000x reference)
import jax
import jax.numpy as jnp
from jax.experimental import pallas as pl


def kernel(indices, table):
    raise NotImplementedError("write your pallas kernel here")


# speedup vs baseline: 1.8625x; 1.8625x over previous
import functools

import jax
import jax.numpy as jnp
from jax import lax
from jax.experimental import pallas as pl
from jax.experimental.pallas import tpu as pltpu
from jax.experimental.pallas import tpu_sc as plsc

_EMBED = 64
_LANES = 16
_NC = 2
_NS = 16
_NW = _NC * _NS
_R = 128
_SPG = 4
_GROUP = _R * _SPG


def _emb_body(n_idx_rows, table_hbm, idx_hbm, out_hbm,
              idx_v, buf0, buf1, flags_v, sg0, sg1, ss0, ss1):
    n_groups = n_idx_rows // _SPG
    per_w = n_idx_rows * _R
    wid = lax.axis_index("s") * _NC + lax.axis_index("c")
    base = wid * per_w

    pltpu.sync_copy(idx_hbm.at[wid], idx_v)

    bufs = (buf0, buf1)
    gsems = (sg0, sg1)
    ssems = (ss0, ss1)
    zeros16 = jnp.zeros((_LANES,), jnp.float32)

    def fire_gather(g, slot):
        for j in range(_SPG):
            pltpu.async_copy(
                table_hbm.at[idx_v.at[g * _SPG + j]],
                bufs[slot].at[pl.ds(j * _R, _R)],
                gsems[slot],
            )

    def wait_gather(g, slot):
        for j in range(_SPG):
            pltpu.make_async_copy(
                table_hbm.at[idx_v.at[g * _SPG + j]],
                bufs[slot].at[pl.ds(j * _R, _R)],
                gsems[slot],
            ).wait()

    def fire_scatter(g, slot):
        pltpu.async_copy(
            bufs[slot], out_hbm.at[pl.ds(base + g * _GROUP, _GROUP)],
            ssems[slot],
        )

    def wait_scatter(g, slot):
        pltpu.make_async_copy(
            bufs[slot], out_hbm.at[pl.ds(base + g * _GROUP, _GROUP)],
            ssems[slot],
        ).wait()

    ones_i = jnp.ones((_LANES,), jnp.int32)
    zeros_i = jnp.zeros((_LANES,), jnp.int32)

    def fixup(g, slot):
        buf = bufs[slot]
        macc = zeros_i
        for j in range(_SPG):
            for k in range(_R // _LANES):
                v = idx_v[g * _SPG + j, pl.ds(k * _LANES, _LANES)]
                macc = macc | jnp.where(v == 0, ones_i, zeros_i)
        any_zero = macc[0]
        for l in range(1, _LANES):
            any_zero = any_zero | macc[l]

        @pl.when(any_zero != 0)
        def _():
            def body(i, carry):
                row = g * _SPG + i // (_R // _LANES)
                col = (i % (_R // _LANES)) * _LANES
                v = idx_v[row, pl.ds(col, _LANES)]
                for l in range(_LANES):
                    s = v[l]

                    @pl.when(s == 0)
                    def _zero_row():
                        r = i * _LANES + l
                        for c in range(_EMBED // _LANES):
                            buf[r, pl.ds(c * _LANES, _LANES)] = zeros16

                return carry

            lax.fori_loop(0, _SPG * (_R // _LANES), body, 0)

    fire_gather(0, 0)
    fire_gather(1, 1)

    n_pairs = n_groups // 2

    def step(i, carry):
        g0 = i * 2
        g1 = g0 + 1
        wait_gather(g0, 0)
        fixup(g0, 0)
        fire_scatter(g0, 0)
        wait_gather(g1, 1)
        fixup(g1, 1)
        fire_scatter(g1, 1)

        @pl.when(i < n_pairs - 1)
        def _():
            wait_scatter(g0, 0)
            fire_gather(g0 + 2, 0)
            wait_scatter(g1, 1)
            fire_gather(g1 + 2, 1)

        return carry

    lax.fori_loop(0, n_pairs, step, 0)
    wait_scatter(n_groups - 2, 0)
    wait_scatter(n_groups - 1, 1)


@functools.lru_cache(maxsize=None)
def _make_emb(vocab, n_idx_rows):
    n = _NW * n_idx_rows * _R
    mesh = plsc.VectorSubcoreMesh(core_axis_name="c", subcore_axis_name="s")
    return pl.kernel(
        functools.partial(_emb_body, n_idx_rows),
        mesh=mesh,
        compiler_params=pltpu.CompilerParams(use_tc_tiling_on_sc=False),
        out_type=jax.ShapeDtypeStruct((n, _EMBED), jnp.float32),
        scratch_types=[
            pltpu.VMEM((n_idx_rows, _R), jnp.int32),
            pltpu.VMEM((_GROUP, _EMBED), jnp.float32),
            pltpu.VMEM((_GROUP, _EMBED), jnp.float32),
            pltpu.VMEM((_LANES,), jnp.int32),
            pltpu.SemaphoreType.DMA,
            pltpu.SemaphoreType.DMA,
            pltpu.SemaphoreType.DMA,
            pltpu.SemaphoreType.DMA,
        ],
    )


def kernel(indices, table):
    b, s = indices.shape
    n = b * s
    n_idx_rows = n // (_NW * _R)
    idx = indices.reshape(_NW, n_idx_rows, _R).astype(jnp.int32)
    out = _make_emb(table.shape[0], n_idx_rows)(table, idx)
    return out.reshape(b, s, _EMBED)


# speedup vs baseline: 1.8792x; 1.0090x over previous
import functools

import jax
import jax.numpy as jnp
from jax import lax
from jax.experimental import pallas as pl
from jax.experimental.pallas import tpu as pltpu
from jax.experimental.pallas import tpu_sc as plsc

_EMBED = 64
_LANES = 16
_NC = 2
_NS = 16
_NW = _NC * _NS
_R = 128
_SLOTS = 8
_AHEAD = 6


def _emb_body(n_groups, table_hbm, idx_hbm, out_hbm, idx_v, buf, sg, ss):
    per_w = n_groups * _R
    wid = lax.axis_index("s") * _NC + lax.axis_index("c")
    base = wid * per_w

    pltpu.sync_copy(idx_hbm.at[wid], idx_v)

    ones_i = jnp.ones((_LANES,), jnp.int32)
    zeros_i = jnp.zeros((_LANES,), jnp.int32)
    zeros_f = jnp.zeros((_LANES,), jnp.float32)

    def fire_gather(g):
        slot = lax.rem(g, _SLOTS)
        pltpu.async_copy(
            table_hbm.at[idx_v.at[g]],
            buf.at[pl.ds(slot * _R, _R)],
            sg,
        )

    def wait_gather(g):
        slot = lax.rem(g, _SLOTS)
        pltpu.make_async_copy(
            table_hbm.at[idx_v.at[g]],
            buf.at[pl.ds(slot * _R, _R)],
            sg,
        ).wait()

    def fire_scatter(g):
        slot = lax.rem(g, _SLOTS)
        pltpu.async_copy(
            buf.at[pl.ds(slot * _R, _R)],
            out_hbm.at[pl.ds(base + g * _R, _R)],
            ss,
        )

    def wait_scatter_one():
        pltpu.make_async_copy(
            buf.at[pl.ds(0, _R)],
            out_hbm.at[pl.ds(base, _R)],
            ss,
        ).wait()

    def fixup(g):
        slot = lax.rem(g, _SLOTS)
        macc = zeros_i
        for k in range(_R // _LANES):
            v = idx_v[g, pl.ds(k * _LANES, _LANES)]
            macc = macc | jnp.where(v == 0, ones_i, zeros_i)
        any_zero = macc[0]
        for l in range(1, _LANES):
            any_zero = any_zero | macc[l]

        @pl.when(any_zero != 0)
        def _():
            def body(k, carry):
                v = idx_v[g, pl.ds(k * _LANES, _LANES)]
                for l in range(_LANES):
                    s = v[l]

                    @pl.when(s == 0)
                    def _zero_row():
                        r = slot * _R + k * _LANES + l
                        for c in range(_EMBED // _LANES):
                            buf[r, pl.ds(c * _LANES, _LANES)] = zeros_f

                return carry

            lax.fori_loop(0, _R // _LANES, body, 0)

    for g in range(_AHEAD):
        fire_gather(g)

    def step(g, carry):
        wait_gather(g)
        fixup(g)
        fire_scatter(g)

        @pl.when(g + _AHEAD < n_groups)
        def _():
            @pl.when(g >= _SLOTS - _AHEAD)
            def _():
                wait_scatter_one()

            fire_gather(g + _AHEAD)

        return carry

    lax.fori_loop(0, n_groups, step, 0)
    n_waited = max(0, (n_groups - _AHEAD) - (_SLOTS - _AHEAD))
    for _ in range(n_groups - n_waited):
        wait_scatter_one()


@functools.lru_cache(maxsize=None)
def _make_emb(vocab, n_groups):
    n = _NW * n_groups * _R
    mesh = plsc.VectorSubcoreMesh(core_axis_name="c", subcore_axis_name="s")
    return pl.kernel(
        functools.partial(_emb_body, n_groups),
        mesh=mesh,
        compiler_params=pltpu.CompilerParams(use_tc_tiling_on_sc=False),
        out_type=jax.ShapeDtypeStruct((n, _EMBED), jnp.float32),
        scratch_types=[
            pltpu.VMEM((n_groups, _R), jnp.int32),
            pltpu.VMEM((_SLOTS * _R, _EMBED), jnp.float32),
            pltpu.SemaphoreType.DMA,
            pltpu.SemaphoreType.DMA,
        ],
    )


def kernel(indices, table):
    b, s = indices.shape
    n = b * s
    n_groups = n // (_NW * _R)
    idx = indices.reshape(_NW, n_groups, _R).astype(jnp.int32)
    out = _make_emb(table.shape[0], n_groups)(table, idx)
    return out.reshape(b, s, _EMBED)
